# baseline (speedup 1.0000x reference)
import jax
import jax.numpy as jnp
from jax.experimental import pallas as pl

F32 = jnp.float32
_KNN = 3
_BIG = 1e30


def _knn_gather(s, sq, h, block_rows=200):
    n = s.shape[0]
    br = block_rows if n % block_rows == 0 else n

    def body(s_ref, sq_ref, st_ref, sqt_ref, sf_ref, h_ref, g_ref):
        hs = jnp.concatenate([h_ref[...], sf_ref[...]], axis=1)
        hs_a = hs.astype(jnp.bfloat16).astype(F32)
        r = hs - hs_a
        hs_b = r.astype(jnp.bfloat16).astype(F32)
        hs_c = r - hs_b
        hs3 = jnp.concatenate([hs_a, hs_b, hs_c], axis=1)

        dot3 = jnp.dot(s_ref[...], st_ref[...], preferred_element_type=F32)
        d2 = (sq_ref[...] + sqt_ref[...]) - 2.0 * dot3
        iota = jax.lax.broadcasted_iota(jnp.int32, d2.shape, 1)
        gs = []
        for k in range(_KNN):
            m = jnp.min(d2, axis=1, keepdims=True)
            idx = jnp.min(jnp.where(d2 == m, iota, n), axis=1, keepdims=True)
            sel = iota == idx
            g3 = jnp.dot(sel.astype(F32), hs3,
                         preferred_element_type=F32)
            gs.append((g3[:, 0:4] + g3[:, 4:8]) + g3[:, 8:12])
            if k < _KNN - 1:
                d2 = jnp.where(sel, _BIG, d2)
        g_ref[...] = jnp.concatenate(gs, axis=1)

    return pl.pallas_call(
        body, grid=(n // br,),
        in_specs=[pl.BlockSpec((br, 3), lambda i: (i, 0)),
                  pl.BlockSpec((br, 1), lambda i: (i, 0)),
                  pl.BlockSpec((3, n), lambda i: (0, 0)),
                  pl.BlockSpec((1, n), lambda i: (0, 0)),
                  pl.BlockSpec((n, 3), lambda i: (0, 0)),
                  pl.BlockSpec((n, 1), lambda i: (0, 0))],
        out_specs=pl.BlockSpec((br, 12), lambda i: (i, 0)),
        out_shape=jax.ShapeDtypeStruct((n, 12), F32),
    )(s, sq, s.T, sq.T, s, h)


def _linear(p, x):
    return x @ p["W"].T + p["b"]


def _gravnet_conv(p, x):
    s = _linear(p["lin_s"], x)
    h = _linear(p["lin_h"], x)
    sq = jnp.sum(s * s, axis=1)
    g12 = _knn_gather(s, sq.reshape(-1, 1), h)
    h_nb = jnp.stack([g12[:, 0], g12[:, 4], g12[:, 8]], axis=1)[..., None]
    s_nb = jnp.stack([g12[:, 1:4], g12[:, 5:8], g12[:, 9:12]], axis=1)
    dist2 = jnp.sum((s[:, None, :] - s_nb) ** 2, axis=-1)
    w = jnp.exp(-10.0 * dist2)
    msg = h_nb * w[..., None]
    agg = jnp.concatenate([jnp.mean(msg, axis=1), jnp.max(msg, axis=1)],
                          axis=-1)
    return x @ p["Wo1"].T + _linear(p["lin_out2"], agg)


def _block(p, x):
    x = x.reshape(x.shape[0], -1)
    x = _linear(p["d1"], x)
    x = jax.nn.relu(_linear(p["d2"], x))
    x = jax.nn.relu(_linear(p["d3"], x))
    for m in ("mp1", "mp2", "mp3", "mp4", "mp5", "mp6"):
        x = _gravnet_conv(p[m], x)
    x = jax.nn.relu(_linear(p["d4"], x))
    x = jax.nn.relu(_linear(p["d5"], x))
    x = jax.nn.relu(_linear(p["d6"], x))
    return x


def kernel(input_hits, params):
    x1 = _block(params["b1"], input_hits)
    x2 = _block(params["b2"], x1)
    x3 = _block(params["b3"], x2)
    x = jnp.concatenate([x1, x2, x3], axis=1)
    x = jax.nn.relu(_linear(params["fc1"], x))
    x = jax.nn.relu(_linear(params["fc2"], x))
    x = jax.nn.relu(_linear(params["fc3"], x))
    mu = jnp.mean(x, axis=0)
    var = jnp.var(x, axis=0)
    x = params["bn_gamma"] * (x - mu) / jnp.sqrt(var + 1e-5) + params["bn_beta"]
    return jax.nn.relu(_linear(params["fc4"], x))


# speedup vs baseline: 1.2819x; 1.2819x over previous
import functools

import jax
import jax.numpy as jnp
from jax import lax
from jax.experimental import pallas as pl
from jax.experimental.pallas import tpu as pltpu
from jax.experimental.pallas import tpu_sc as plsc

F32 = jnp.float32
_KNN = 3
_BIG = 1e30


def _knn_idx(s, sq, block_rows=400):
    n = s.shape[0]
    br = block_rows if n % block_rows == 0 else n

    def body(s_ref, sq_ref, st_ref, sqt_ref, idx_ref):
        dot3 = jnp.dot(s_ref[...], st_ref[...], preferred_element_type=F32)
        d2 = (sq_ref[...] + sqt_ref[...]) - 2.0 * dot3
        iota = jax.lax.broadcasted_iota(jnp.int32, d2.shape, 1)
        idxs = []
        for k in range(_KNN):
            m = jnp.min(d2, axis=1, keepdims=True)
            idx = jnp.min(jnp.where(d2 == m, iota, n), axis=1, keepdims=True)
            idxs.append(idx)
            if k < _KNN - 1:
                d2 = jnp.where(iota == idx, _BIG, d2)
        idx_ref[...] = jnp.concatenate(idxs, axis=1)

    return pl.pallas_call(
        body, grid=(n // br,),
        in_specs=[pl.BlockSpec((br, 3), lambda i: (i, 0)),
                  pl.BlockSpec((br, 1), lambda i: (i, 0)),
                  pl.BlockSpec((3, n), lambda i: (0, 0)),
                  pl.BlockSpec((1, n), lambda i: (0, 0))],
        out_specs=pl.BlockSpec((br, _KNN), lambda i: (i, 0)),
        out_shape=jax.ShapeDtypeStruct((n, _KNN), jnp.int32),
    )(s, sq, s.T, sq.T)


def _sc_gather(table, idx_flat):
    b = idx_flat.shape[0]
    info = plsc.get_sparse_core_info()
    nc, ns = info.num_cores, info.num_subcores
    nw = nc * ns
    nchunk = 2
    b_per_c = b // (nw * nchunk)
    mesh = plsc.VectorSubcoreMesh(core_axis_name="c", subcore_axis_name="s")

    @functools.partial(
        pl.kernel, mesh=mesh,
        out_type=jax.ShapeDtypeStruct((b, 128), F32),
        scratch_types=[pltpu.VMEM((b_per_c,), jnp.int32),
                       pltpu.VMEM((b_per_c, 128), F32),
                       pltpu.SemaphoreType.DMA],
    )
    def k(table_hbm, idx_hbm, out_hbm, idx_v, rows_v, sem):
        wid = lax.axis_index("s") * nc + lax.axis_index("c")
        for c in range(nchunk):
            base = (wid * nchunk + c) * b_per_c
            pltpu.sync_copy(idx_hbm.at[pl.ds(base, b_per_c)], idx_v)
            pltpu.async_copy(table_hbm.at[idx_v], rows_v, sem).wait()
            pltpu.sync_copy(rows_v, out_hbm.at[pl.ds(base, b_per_c)])

    return k(table, idx_flat)


def _linear(p, x):
    return x @ p["W"].T + p["b"]


def _gravnet_conv(p, x):
    n = x.shape[0]
    s = _linear(p["lin_s"], x)
    h = _linear(p["lin_h"], x)
    sq = jnp.sum(s * s, axis=1)
    idx = _knn_idx(s, sq.reshape(-1, 1))
    table = jnp.concatenate([h, s, jnp.zeros((n, 124), F32)], axis=1)
    b = n * _KNN
    b_pad = (-b) % 512
    idx_flat = jnp.concatenate(
        [idx.reshape(-1), jnp.zeros((b_pad,), jnp.int32)])
    rows = _sc_gather(table, idx_flat)[:b].reshape(n, _KNN, 128)
    h_nb = rows[:, :, 0:1]
    s_nb = rows[:, :, 1:4]
    dist2 = jnp.sum((s[:, None, :] - s_nb) ** 2, axis=-1)
    w = jnp.exp(-10.0 * dist2)
    msg = h_nb * w[..., None]
    agg = jnp.concatenate([jnp.mean(msg, axis=1), jnp.max(msg, axis=1)],
                          axis=-1)
    return x @ p["Wo1"].T + _linear(p["lin_out2"], agg)


def _block(p, x):
    x = x.reshape(x.shape[0], -1)
    x = _linear(p["d1"], x)
    x = jax.nn.relu(_linear(p["d2"], x))
    x = jax.nn.relu(_linear(p["d3"], x))
    for m in ("mp1", "mp2", "mp3", "mp4", "mp5", "mp6"):
        x = _gravnet_conv(p[m], x)
    x = jax.nn.relu(_linear(p["d4"], x))
    x = jax.nn.relu(_linear(p["d5"], x))
    x = jax.nn.relu(_linear(p["d6"], x))
    return x


def kernel(input_hits, params):
    x1 = _block(params["b1"], input_hits)
    x2 = _block(params["b2"], x1)
    x3 = _block(params["b3"], x2)
    x = jnp.concatenate([x1, x2, x3], axis=1)
    x = jax.nn.relu(_linear(params["fc1"], x))
    x = jax.nn.relu(_linear(params["fc2"], x))
    x = jax.nn.relu(_linear(params["fc3"], x))
    mu = jnp.mean(x, axis=0)
    var = jnp.var(x, axis=0)
    x = params["bn_gamma"] * (x - mu) / jnp.sqrt(var + 1e-5) + params["bn_beta"]
    return jax.nn.relu(_linear(params["fc4"], x))
